# baseline (speedup 1.0000x reference)
import functools

import jax
import jax.numpy as jnp
from jax import lax
from jax.experimental import pallas as pl
from jax.experimental.pallas import tpu as pltpu
from jax.experimental.pallas import tpu_sc as plsc

_L = 16


@functools.lru_cache(maxsize=None)
def _num_masked(batch, seq):
    import numpy as np

    with jax.ensure_compile_time_eval():
        n = jax.random.randint(jax.random.key(42), (batch,), 1, seq + 1)
        return tuple(int(x) for x in np.asarray(n))


@functools.lru_cache(maxsize=None)
def _make_mask_kernel(batch, seq):
    n_const = _num_masked(batch, seq)
    mesh = plsc.VectorSubcoreMesh(
        core_axis_name="c", subcore_axis_name="s", num_cores=1,
        num_subcores=batch,
    )

    @functools.partial(
        pl.kernel,
        mesh=mesh,
        out_type=jax.ShapeDtypeStruct((batch, seq), jnp.float32),
        compiler_params=pltpu.CompilerParams(needs_layout_passes=False),
        scratch_types=[
            pltpu.VMEM((seq,), jnp.int32),
            pltpu.VMEM((seq,), jnp.float32),
            pltpu.SemaphoreType.DMA,
        ],
    )
    def mask_kernel(orders_hbm, out_hbm, idx_v, row_v, sem):
        wid = lax.axis_index("s") + lax.axis_index("c")
        half = seq // 2

        @pl.when(wid < batch)
        def _():
            cp_a = pltpu.async_copy(
                orders_hbm.at[wid, pl.ds(0, half)], idx_v.at[pl.ds(0, half)], sem
            )
            cp_b = pltpu.async_copy(
                orders_hbm.at[wid, pl.ds(half, half)],
                idx_v.at[pl.ds(half, half)],
                sem,
            )
            nw = jnp.int32(n_const[0])
            for b in range(1, batch):
                nw = jnp.where(wid == b, jnp.int32(n_const[b]), nw)
            jbase = lax.iota(jnp.int32, 16)

            cp_a.wait()

            @plsc.parallel_loop(0, half, step=_L, unroll=4)
            def _body_a(j0):
                idx16 = idx_v[pl.ds(j0, _L)]
                vals = jnp.where(jbase + j0 < nw, 1.0, 0.0)
                plsc.store_scatter(row_v, [idx16], vals)

            cp_b.wait()

            @plsc.parallel_loop(half, seq, step=_L, unroll=4)
            def _body_b(j0):
                idx16 = idx_v[pl.ds(j0, _L)]
                vals = jnp.where(jbase + j0 < nw, 1.0, 0.0)
                plsc.store_scatter(row_v, [idx16], vals)

            pltpu.sync_copy(row_v, out_hbm.at[wid])

    return mask_kernel


def kernel(patches, orders):
    batch, seq, _ = patches.shape
    idx = orders.astype(jnp.int32)
    return _make_mask_kernel(batch, seq)(idx)


# speedup vs baseline: 1.0068x; 1.0068x over previous
import functools

import jax
import jax.numpy as jnp
from jax import lax
from jax.experimental import pallas as pl
from jax.experimental.pallas import tpu as pltpu
from jax.experimental.pallas import tpu_sc as plsc

_L = 16


@functools.lru_cache(maxsize=None)
def _num_masked(batch, seq):
    import numpy as np

    with jax.ensure_compile_time_eval():
        n = jax.random.randint(jax.random.key(42), (batch,), 1, seq + 1)
        return tuple(int(x) for x in np.asarray(n))


@functools.lru_cache(maxsize=None)
def _make_mask_kernel(batch, seq):
    n_const = _num_masked(batch, seq)
    mesh = plsc.VectorSubcoreMesh(
        core_axis_name="c", subcore_axis_name="s", num_cores=1
    )

    @functools.partial(
        pl.kernel,
        mesh=mesh,
        out_type=jax.ShapeDtypeStruct((batch, seq), jnp.float32),
        compiler_params=pltpu.CompilerParams(needs_layout_passes=False),
        scratch_types=[
            pltpu.VMEM((seq,), jnp.int32),
            pltpu.VMEM((seq,), jnp.float32),
            pltpu.SemaphoreType.DMA,
        ],
    )
    def mask_kernel(orders_hbm, out_hbm, idx_v, row_v, sem):
        wid = lax.axis_index("s") + lax.axis_index("c")
        half = seq // 2

        @pl.when(wid < batch)
        def _():
            cp_a = pltpu.async_copy(
                orders_hbm.at[wid, pl.ds(0, half)], idx_v.at[pl.ds(0, half)], sem
            )
            cp_b = pltpu.async_copy(
                orders_hbm.at[wid, pl.ds(half, half)],
                idx_v.at[pl.ds(half, half)],
                sem,
            )
            nw = jnp.int32(n_const[0])
            for b in range(1, batch):
                nw = jnp.where(wid == b, jnp.int32(n_const[b]), nw)
            jbase = lax.iota(jnp.int32, 16)

            cp_a.wait()

            @plsc.parallel_loop(0, half, step=_L, unroll=16)
            def _body_a(j0):
                idx16 = idx_v[pl.ds(j0, _L)]
                vals = jnp.where(jbase + j0 < nw, 1.0, 0.0)
                plsc.store_scatter(row_v, [idx16], vals)

            cp_b.wait()

            @plsc.parallel_loop(half, seq, step=_L, unroll=16)
            def _body_b(j0):
                idx16 = idx_v[pl.ds(j0, _L)]
                vals = jnp.where(jbase + j0 < nw, 1.0, 0.0)
                plsc.store_scatter(row_v, [idx16], vals)

            pltpu.sync_copy(row_v, out_hbm.at[wid])

    return mask_kernel


def kernel(patches, orders):
    batch, seq, _ = patches.shape
    idx = orders.astype(jnp.int32)
    return _make_mask_kernel(batch, seq)(idx)
